# Initial kernel scaffold; baseline (speedup 1.0000x reference)
import jax
import jax.numpy as jnp
from jax.experimental import pallas as pl


def kernel(query, value, reference_points, spatial_shapes, W_off, b_off, W_attn, b_attn, W_val, b_val, W_out, b_out):
    raise NotImplementedError("write your pallas kernel here")


# speedup vs baseline: 23.8207x; 23.8207x over previous
import functools

import jax
import jax.numpy as jnp
from jax import lax
from jax.experimental import pallas as pl
from jax.experimental.pallas import tpu as pltpu
from jax.experimental.pallas import tpu_sc as plsc

NH = 8
NP = 8
NZ = 4
HD = 32
EMB = 256
GRID_HW = 100
NQP = 10240
BQA = 512
BQC = 128


def _vproj_body(x_ref, wt_ref, b_ref, o_ref):
    o_ref[...] = (
        jnp.dot(x_ref[...], wt_ref[...], preferred_element_type=jnp.float32)
        + b_ref[...]
    )


def _vproj(x, w_t, b):
    m = x.shape[0]
    bm = 2000
    return pl.pallas_call(
        _vproj_body,
        grid=(m // bm,),
        in_specs=[
            pl.BlockSpec((bm, EMB), lambda i: (i, 0)),
            pl.BlockSpec((EMB, EMB), lambda i: (0, 0)),
            pl.BlockSpec((1, EMB), lambda i: (0, 0)),
        ],
        out_specs=pl.BlockSpec((bm, EMB), lambda i: (i, 0)),
        out_shape=jax.ShapeDtypeStruct((m, EMB), jnp.float32),
    )(x, w_t, b)


def _addr_body(qt_ref, refx_ref, refy_ref, wx_ref, bx_ref, wy_ref, by_ref,
               wa_ref, ba_ref, w_ref, idx_ref):
    b = pl.program_id(0)
    qt = qt_ref[...]
    offx = (
        jnp.dot(wx_ref[...], qt, preferred_element_type=jnp.float32)
        + bx_ref[...]
    )
    offy = (
        jnp.dot(wy_ref[...], qt, preferred_element_type=jnp.float32)
        + by_ref[...]
    )
    att = (
        jnp.dot(wa_ref[...], qt, preferred_element_type=jnp.float32)
        + ba_ref[...]
    )
    att3 = att.reshape(NH, NP, att.shape[-1])
    att3 = att3 - jnp.max(att3, axis=1, keepdims=True)
    att3 = jnp.exp(att3)
    att3 = att3 / jnp.sum(att3, axis=1, keepdims=True)
    att = att3.reshape(NH * NP, att.shape[-1])

    refx = jnp.broadcast_to(refx_ref[0][None], (NH * NP // NZ, NZ, BQA))
    refy = jnp.broadcast_to(refy_ref[0][None], (NH * NP // NZ, NZ, BQA))
    x = (refx.reshape(NH * NP, BQA) + offx) * float(GRID_HW) - 0.5
    y = (refy.reshape(NH * NP, BQA) + offy) * float(GRID_HW) - 0.5
    x0 = jnp.floor(x)
    y0 = jnp.floor(y)
    hrow = lax.broadcasted_iota(jnp.int32, (NH * NP, BQA), 0) // NP
    rowbase = (b * NH + hrow) * (GRID_HW * GRID_HW)
    ws = []
    idxs = []
    for dx, dy in ((0, 0), (1, 0), (0, 1), (1, 1)):
        xi = x0 + dx
        yi = y0 + dy
        wgt = (1.0 - jnp.abs(x - xi)) * (1.0 - jnp.abs(y - yi))
        valid = (
            (xi >= 0.0) & (xi < float(GRID_HW))
            & (yi >= 0.0) & (yi < float(GRID_HW))
        )
        ws.append(wgt * valid.astype(jnp.float32) * att)
        xic = jnp.clip(xi, 0.0, float(GRID_HW - 1)).astype(jnp.int32)
        yic = jnp.clip(yi, 0.0, float(GRID_HW - 1)).astype(jnp.int32)
        idxs.append(rowbase + yic * GRID_HW + xic)
    w_ref[0] = jnp.concatenate(ws, axis=0)
    idx_ref[0] = jnp.concatenate(idxs, axis=0)


def _addresses(qt, refxt, refyt, wx, bx, wy, by, wa, ba):
    nhist = refxt.shape[0]
    return pl.pallas_call(
        _addr_body,
        grid=(nhist, NQP // BQA),
        in_specs=[
            pl.BlockSpec((EMB, BQA), lambda b, i: (0, i)),
            pl.BlockSpec((1, NZ, BQA), lambda b, i: (b, 0, i)),
            pl.BlockSpec((1, NZ, BQA), lambda b, i: (b, 0, i)),
            pl.BlockSpec((NH * NP, EMB), lambda b, i: (0, 0)),
            pl.BlockSpec((NH * NP, 1), lambda b, i: (0, 0)),
            pl.BlockSpec((NH * NP, EMB), lambda b, i: (0, 0)),
            pl.BlockSpec((NH * NP, 1), lambda b, i: (0, 0)),
            pl.BlockSpec((NH * NP, EMB), lambda b, i: (0, 0)),
            pl.BlockSpec((NH * NP, 1), lambda b, i: (0, 0)),
        ],
        out_specs=[
            pl.BlockSpec((1, NH * NP * 4, BQA), lambda b, i: (b, 0, i)),
            pl.BlockSpec((1, NH * NP * 4, BQA), lambda b, i: (b, 0, i)),
        ],
        out_shape=[
            jax.ShapeDtypeStruct((nhist, NH * NP * 4, NQP), jnp.float32),
            jax.ShapeDtypeStruct((nhist, NH * NP * 4, NQP), jnp.int32),
        ],
    )(qt, refxt, refyt, wx, bx, wy, by, wa, ba)


def _sc_gather(vtab, idx_flat):
    nrows = idx_flat.shape[0]
    try:
        info = plsc.get_sparse_core_info()
        nc, ns = info.num_cores, info.num_subcores
    except Exception:
        nc, ns = 2, 16
    nw = nc * ns
    per_w = nrows // nw
    chunk = 128
    iters = per_w // chunk
    mesh = plsc.VectorSubcoreMesh(core_axis_name="c", subcore_axis_name="s")

    @functools.partial(
        pl.kernel,
        mesh=mesh,
        compiler_params=pltpu.CompilerParams(use_tc_tiling_on_sc=False),
        out_type=jax.ShapeDtypeStruct((nrows, HD), jnp.float32),
        scratch_types=[
            pltpu.VMEM((chunk,), jnp.int32),
            pltpu.VMEM((chunk, HD), jnp.float32),
            pltpu.SemaphoreType.DMA,
        ],
    )
    def gather_kernel(vtab_hbm, idx_hbm, out_hbm, idx_v, rows_v, sem):
        wid = lax.axis_index("s") * nc + lax.axis_index("c")
        base = wid * per_w

        def body(i, carry):
            off = base + i * chunk
            pltpu.sync_copy(idx_hbm.at[pl.ds(off, chunk)], idx_v)
            pltpu.async_copy(vtab_hbm.at[idx_v], rows_v, sem).wait()
            pltpu.sync_copy(rows_v, out_hbm.at[pl.ds(off, chunk)])
            return carry

        lax.fori_loop(0, iters, body, 0)

    return gather_kernel(vtab, idx_flat)


def _combine_body(g_ref, w_ref, e_ref, qt_ref, wout_ref, bout_ref, o_ref):
    nhist = w_ref.shape[0]
    accs = []
    for bh in range(nhist):
        wexp = jnp.dot(w_ref[bh], e_ref[...], preferred_element_type=jnp.float32)
        prod = g_ref[bh] * wexp
        t = prod.reshape(4, NH * NP, BQC * HD).sum(axis=0)
        t = t.reshape(NH, NP, BQC * HD).sum(axis=1)
        accs.append(t)
    acc = (accs[0] + accs[1]) * 0.5
    a_t = acc.reshape(NH, BQC, HD)
    a_t = jnp.transpose(a_t, (0, 2, 1)).reshape(EMB, BQC)
    out_t = (
        jnp.dot(wout_ref[...], a_t, preferred_element_type=jnp.float32)
        + bout_ref[...]
    )
    o_ref[...] = out_t + qt_ref[...]


def _combine(g3, w3, expand, qt, wout, bout):
    nhist = w3.shape[0]
    return pl.pallas_call(
        _combine_body,
        grid=(NQP // BQC,),
        in_specs=[
            pl.BlockSpec((nhist, NH * NP * 4, BQC * HD), lambda i: (0, 0, i)),
            pl.BlockSpec((nhist, NH * NP * 4, BQC), lambda i: (0, 0, i)),
            pl.BlockSpec((BQC, BQC * HD), lambda i: (0, 0)),
            pl.BlockSpec((EMB, BQC), lambda i: (0, i)),
            pl.BlockSpec((EMB, EMB), lambda i: (0, 0)),
            pl.BlockSpec((EMB, 1), lambda i: (0, 0)),
        ],
        out_specs=pl.BlockSpec((EMB, BQC), lambda i: (0, i)),
        out_shape=jax.ShapeDtypeStruct((EMB, NQP), jnp.float32),
    )(g3, w3, expand, qt, wout, bout)


def kernel(query, value, reference_points, spatial_shapes,
           W_off, b_off, W_attn, b_attn, W_val, b_val, W_out, b_out):
    bs, nq, emb = query.shape
    nhist = value.shape[0]
    pad = NQP - nq

    ssf = spatial_shapes.astype(jnp.float32)
    wx = W_off[0::2] / ssf[0, 1]
    wy = W_off[1::2] / ssf[0, 0]
    bx = (b_off[0::2] / ssf[0, 1]).reshape(-1, 1)
    by = (b_off[1::2] / ssf[0, 0]).reshape(-1, 1)
    wa = W_attn
    ba = b_attn.reshape(-1, 1)

    qt = jnp.pad(query.reshape(nq, emb).T, ((0, 0), (0, pad)))
    refxt = jnp.pad(reference_points[..., 0].transpose(0, 2, 1),
                    ((0, 0), (0, 0), (0, pad)))
    refyt = jnp.pad(reference_points[..., 1].transpose(0, 2, 1),
                    ((0, 0), (0, 0), (0, pad)))

    v = _vproj(value.reshape(nhist * nq, emb), W_val.T, b_val.reshape(1, -1))
    vtab = (
        v.reshape(nhist, nq, NH, HD)
        .transpose(0, 2, 1, 3)
        .reshape(nhist * NH * nq, HD)
    )

    w3, idx3 = _addresses(qt, refxt, refyt, wx, bx, wy, by, wa, ba)

    g = _sc_gather(vtab, idx3.reshape(-1))

    g3 = g.reshape(nhist, NH * NP * 4, NQP * HD)
    expand = jnp.broadcast_to(
        jnp.eye(BQC, dtype=jnp.float32)[:, :, None], (BQC, BQC, HD)
    ).reshape(BQC, BQC * HD)
    out_t = _combine(g3, w3, expand, qt, W_out, b_out.reshape(-1, 1))
    return out_t[:, :nq].T.reshape(bs, nq, emb)


# speedup vs baseline: 28.6534x; 1.2029x over previous
import functools

import jax
import jax.numpy as jnp
from jax import lax
from jax.experimental import pallas as pl
from jax.experimental.pallas import tpu as pltpu
from jax.experimental.pallas import tpu_sc as plsc

NH = 8
NP = 8
NZ = 4
HD = 32
EMB = 256
GRID_HW = 100
NQP = 10240
BQA = 512
BQC = 128


def _vproj_body(x_ref, wt_ref, b_ref, o_ref):
    o_ref[...] = (
        jnp.dot(x_ref[...], wt_ref[...], preferred_element_type=jnp.float32)
        + b_ref[...]
    )


def _vproj(x, w_t, b):
    m = x.shape[0]
    bm = 2000
    return pl.pallas_call(
        _vproj_body,
        grid=(m // bm,),
        in_specs=[
            pl.BlockSpec((bm, EMB), lambda i: (i, 0)),
            pl.BlockSpec((EMB, EMB), lambda i: (0, 0)),
            pl.BlockSpec((1, EMB), lambda i: (0, 0)),
        ],
        out_specs=pl.BlockSpec((bm, EMB), lambda i: (i, 0)),
        out_shape=jax.ShapeDtypeStruct((m, EMB), jnp.float32),
    )(x, w_t, b)


def _addr_body(qt_ref, refx_ref, refy_ref, wx_ref, bx_ref, wy_ref, by_ref,
               wa_ref, ba_ref, w_ref, idx_ref):
    b = pl.program_id(0)
    qt = qt_ref[...]
    offx = (
        jnp.dot(wx_ref[...], qt, preferred_element_type=jnp.float32)
        + bx_ref[...]
    )
    offy = (
        jnp.dot(wy_ref[...], qt, preferred_element_type=jnp.float32)
        + by_ref[...]
    )
    att = (
        jnp.dot(wa_ref[...], qt, preferred_element_type=jnp.float32)
        + ba_ref[...]
    )
    att3 = att.reshape(NH, NP, att.shape[-1])
    att3 = att3 - jnp.max(att3, axis=1, keepdims=True)
    att3 = jnp.exp(att3)
    att3 = att3 / jnp.sum(att3, axis=1, keepdims=True)
    att = att3.reshape(NH * NP, att.shape[-1])

    refx = jnp.broadcast_to(refx_ref[0][None], (NH * NP // NZ, NZ, BQA))
    refy = jnp.broadcast_to(refy_ref[0][None], (NH * NP // NZ, NZ, BQA))
    x = (refx.reshape(NH * NP, BQA) + offx) * float(GRID_HW) - 0.5
    y = (refy.reshape(NH * NP, BQA) + offy) * float(GRID_HW) - 0.5
    x0 = jnp.floor(x)
    y0 = jnp.floor(y)
    hrow = lax.broadcasted_iota(jnp.int32, (NH * NP, BQA), 0) // NP
    rowbase = (b * NH + hrow) * (GRID_HW * GRID_HW)
    ws = []
    idxs = []
    for dx, dy in ((0, 0), (1, 0), (0, 1), (1, 1)):
        xi = x0 + dx
        yi = y0 + dy
        wgt = (1.0 - jnp.abs(x - xi)) * (1.0 - jnp.abs(y - yi))
        valid = (
            (xi >= 0.0) & (xi < float(GRID_HW))
            & (yi >= 0.0) & (yi < float(GRID_HW))
        )
        ws.append(wgt * valid.astype(jnp.float32) * att)
        xic = jnp.clip(xi, 0.0, float(GRID_HW - 1)).astype(jnp.int32)
        yic = jnp.clip(yi, 0.0, float(GRID_HW - 1)).astype(jnp.int32)
        idxs.append(rowbase + yic * GRID_HW + xic)
    w_ref[0] = jnp.concatenate(ws, axis=0)
    idx_ref[0] = jnp.concatenate(idxs, axis=0)


def _addresses(qt, refxt, refyt, wx, bx, wy, by, wa, ba):
    nhist = refxt.shape[0]
    return pl.pallas_call(
        _addr_body,
        grid=(nhist, NQP // BQA),
        in_specs=[
            pl.BlockSpec((EMB, BQA), lambda b, i: (0, i)),
            pl.BlockSpec((1, NZ, BQA), lambda b, i: (b, 0, i)),
            pl.BlockSpec((1, NZ, BQA), lambda b, i: (b, 0, i)),
            pl.BlockSpec((NH * NP, EMB), lambda b, i: (0, 0)),
            pl.BlockSpec((NH * NP, 1), lambda b, i: (0, 0)),
            pl.BlockSpec((NH * NP, EMB), lambda b, i: (0, 0)),
            pl.BlockSpec((NH * NP, 1), lambda b, i: (0, 0)),
            pl.BlockSpec((NH * NP, EMB), lambda b, i: (0, 0)),
            pl.BlockSpec((NH * NP, 1), lambda b, i: (0, 0)),
        ],
        out_specs=[
            pl.BlockSpec((1, NH * NP * 4, BQA), lambda b, i: (b, 0, i)),
            pl.BlockSpec((1, NH * NP * 4, BQA), lambda b, i: (b, 0, i)),
        ],
        out_shape=[
            jax.ShapeDtypeStruct((nhist, NH * NP * 4, NQP), jnp.float32),
            jax.ShapeDtypeStruct((nhist, NH * NP * 4, NQP), jnp.int32),
        ],
    )(qt, refxt, refyt, wx, bx, wy, by, wa, ba)


def _sc_gather(vtab, idx_flat):
    nrows = idx_flat.shape[0]
    try:
        info = plsc.get_sparse_core_info()
        nc, ns = info.num_cores, info.num_subcores
    except Exception:
        nc, ns = 2, 16
    nw = nc * ns
    per_w = nrows // nw
    chunk = 128
    nf = 4
    iters = per_w // (chunk * nf)
    mesh = plsc.VectorSubcoreMesh(core_axis_name="c", subcore_axis_name="s")

    @functools.partial(
        pl.kernel,
        mesh=mesh,
        compiler_params=pltpu.CompilerParams(use_tc_tiling_on_sc=False),
        out_type=jax.ShapeDtypeStruct((nrows, HD), jnp.float32),
        scratch_types=[
            pltpu.VMEM((nf, chunk), jnp.int32),
            pltpu.VMEM((nf, chunk, HD), jnp.float32),
            pltpu.SemaphoreType.DMA,
        ],
    )
    def gather_kernel(vtab_hbm, idx_hbm, out_hbm, idx_v, rows_v, sem):
        wid = lax.axis_index("s") * nc + lax.axis_index("c")
        base = wid * per_w

        def body(i, carry):
            off = base + i * (chunk * nf)
            for j in range(nf):
                pltpu.sync_copy(idx_hbm.at[pl.ds(off + j * chunk, chunk)],
                                idx_v.at[j])
            copies = [
                pltpu.async_copy(vtab_hbm.at[idx_v.at[j]], rows_v.at[j], sem)
                for j in range(nf)
            ]
            for c in copies:
                c.wait()
            for j in range(nf):
                pltpu.sync_copy(rows_v.at[j],
                                out_hbm.at[pl.ds(off + j * chunk, chunk)])
            return carry

        lax.fori_loop(0, iters, body, 0)

    return gather_kernel(vtab, idx_flat)


def _combine_body(g_ref, w_ref, e_ref, qt_ref, wout_ref, bout_ref, o_ref):
    nhist = w_ref.shape[0]
    accs = []
    for bh in range(nhist):
        wexp = jnp.dot(w_ref[bh], e_ref[...], preferred_element_type=jnp.float32)
        prod = g_ref[bh] * wexp
        t = prod.reshape(4, NH * NP, BQC * HD).sum(axis=0)
        t = t.reshape(NH, NP, BQC * HD).sum(axis=1)
        accs.append(t)
    acc = (accs[0] + accs[1]) * 0.5
    a_t = acc.reshape(NH, BQC, HD)
    a_t = jnp.transpose(a_t, (0, 2, 1)).reshape(EMB, BQC)
    out_t = (
        jnp.dot(wout_ref[...], a_t, preferred_element_type=jnp.float32)
        + bout_ref[...]
    )
    o_ref[...] = out_t + qt_ref[...]


def _combine(g3, w3, expand, qt, wout, bout):
    nhist = w3.shape[0]
    return pl.pallas_call(
        _combine_body,
        grid=(NQP // BQC,),
        in_specs=[
            pl.BlockSpec((nhist, NH * NP * 4, BQC * HD), lambda i: (0, 0, i)),
            pl.BlockSpec((nhist, NH * NP * 4, BQC), lambda i: (0, 0, i)),
            pl.BlockSpec((BQC, BQC * HD), lambda i: (0, 0)),
            pl.BlockSpec((EMB, BQC), lambda i: (0, i)),
            pl.BlockSpec((EMB, EMB), lambda i: (0, 0)),
            pl.BlockSpec((EMB, 1), lambda i: (0, 0)),
        ],
        out_specs=pl.BlockSpec((EMB, BQC), lambda i: (0, i)),
        out_shape=jax.ShapeDtypeStruct((EMB, NQP), jnp.float32),
    )(g3, w3, expand, qt, wout, bout)


def kernel(query, value, reference_points, spatial_shapes,
           W_off, b_off, W_attn, b_attn, W_val, b_val, W_out, b_out):
    bs, nq, emb = query.shape
    nhist = value.shape[0]
    pad = NQP - nq

    ssf = spatial_shapes.astype(jnp.float32)
    wx = W_off[0::2] / ssf[0, 1]
    wy = W_off[1::2] / ssf[0, 0]
    bx = (b_off[0::2] / ssf[0, 1]).reshape(-1, 1)
    by = (b_off[1::2] / ssf[0, 0]).reshape(-1, 1)
    wa = W_attn
    ba = b_attn.reshape(-1, 1)

    qt = jnp.pad(query.reshape(nq, emb).T, ((0, 0), (0, pad)))
    refxt = jnp.pad(reference_points[..., 0].transpose(0, 2, 1),
                    ((0, 0), (0, 0), (0, pad)))
    refyt = jnp.pad(reference_points[..., 1].transpose(0, 2, 1),
                    ((0, 0), (0, 0), (0, pad)))

    v = _vproj(value.reshape(nhist * nq, emb), W_val.T, b_val.reshape(1, -1))
    vtab = (
        v.reshape(nhist, nq, NH, HD)
        .transpose(0, 2, 1, 3)
        .reshape(nhist * NH * nq, HD)
    )

    w3, idx3 = _addresses(qt, refxt, refyt, wx, bx, wy, by, wa, ba)

    g = _sc_gather(vtab, idx3.reshape(-1))

    g3 = g.reshape(nhist, NH * NP * 4, NQP * HD)
    expand = jnp.broadcast_to(
        jnp.eye(BQC, dtype=jnp.float32)[:, :, None], (BQC, BQC, HD)
    ).reshape(BQC, BQC * HD)
    out_t = _combine(g3, w3, expand, qt, W_out, b_out.reshape(-1, 1))
    return out_t[:, :nq].T.reshape(bs, nq, emb)
